# baseline (speedup 1.0000x reference)
import functools

import jax
import jax.numpy as jnp
from jax.experimental import pallas as pl
from jax.experimental.pallas import tpu as pltpu

C = 64
H = W = 28
HW = H * W
LPAD = 128
SEG = 1024
G = 8


def _conv_body(first, x_ref, w_ref, s_ref, t_ref, out_ref, stat_ref, patch_ref):
    step = pl.program_id(0)

    @pl.when(step == 0)
    def _init():
        stat_ref[...] = jnp.zeros_like(stat_ref)
        patch_ref[...] = jnp.zeros_like(patch_ref)

    if first:
        xn = x_ref[...]
    else:
        x = x_ref[...].astype(jnp.float32)
        xn = jnp.maximum(x * s_ref[...][None] + t_ref[...][None], 0.0)

    col = jax.lax.broadcasted_iota(jnp.int32, (1, HW), 1) % W
    m_left = jnp.where(col == W - 1, 0.0, 1.0)
    m_right = jnp.where(col == 0, 0.0, 1.0)
    xb = xn.astype(jnp.bfloat16)
    xb_l = (xn * m_left).astype(jnp.bfloat16)
    xb_r = (xn * m_right).astype(jnp.bfloat16)

    for g in range(G):
        base = g * SEG + LPAD
        srcs = (xb_l[g], xb[g], xb_r[g])
        for ky in range(3):
            for kx in range(3):
                off = (ky - 1) * W + (kx - 1)
                patch_ref[ky * 3 + kx, :, base - off:base - off + HW] = srcs[kx]

    xs = patch_ref[...].reshape(9 * C, G * SEG)
    y = (jnp.dot(w_ref[0], xs, preferred_element_type=jnp.float32)
         + jnp.dot(w_ref[1], xs, preferred_element_type=jnp.float32))

    pos = jax.lax.broadcasted_iota(jnp.int32, (1, G * SEG), 1) % SEG
    y = y * jnp.where((pos >= LPAD) & (pos < LPAD + HW), 1.0, 0.0)
    stat_ref[0:1, :] = stat_ref[0:1, :] + jnp.sum(y, axis=1)[None]
    stat_ref[1:2, :] = stat_ref[1:2, :] + jnp.sum(y * y, axis=1)[None]
    for g in range(G):
        base = g * SEG + LPAD
        out_ref[g] = y[:, base:base + HW]


def _conv_layer(x, w576, s, t, first):
    batch = x.shape[0]
    return pl.pallas_call(
        functools.partial(_conv_body, first),
        grid=(batch // G,),
        in_specs=[
            pl.BlockSpec((G, C, HW), lambda i: (i, 0, 0)),
            pl.BlockSpec((2, C, 9 * C), lambda i: (0, 0, 0)),
            pl.BlockSpec((C, 1), lambda i: (0, 0)),
            pl.BlockSpec((C, 1), lambda i: (0, 0)),
        ],
        out_specs=[pl.BlockSpec((G, C, HW), lambda i: (i, 0, 0)),
                   pl.BlockSpec((8, C), lambda i: (0, 0))],
        out_shape=[jax.ShapeDtypeStruct((batch, C, HW), jnp.float32),
                   jax.ShapeDtypeStruct((8, C), jnp.float32)],
        scratch_shapes=[pltpu.VMEM((9, C, G * SEG), jnp.bfloat16)],
    )(x, w576, s, t)


def _fc_body(x_ref, w_ref, s_ref, t_ref, out_ref, stat_ref):
    step = pl.program_id(0)

    @pl.when(step == 0)
    def _init():
        stat_ref[...] = jnp.zeros_like(stat_ref)

    x = x_ref[...].astype(jnp.float32)
    xn = jnp.maximum(x * s_ref[...][None] + t_ref[...][None], 0.0)
    w = w_ref[...]
    ssum = None
    ssq = None
    for g in range(G):
        e = jnp.dot(w, xn[g], preferred_element_type=jnp.float32)
        out_ref[g] = e
        ps = jnp.sum(e, axis=1)
        pq = jnp.sum(e * e, axis=1)
        ssum = ps if ssum is None else ssum + ps
        ssq = pq if ssq is None else ssq + pq
    stat_ref[0:1, :] = stat_ref[0:1, :] + ssum[None]
    stat_ref[1:2, :] = stat_ref[1:2, :] + ssq[None]


def _fc_layer(x, w, s, t):
    batch = x.shape[0]
    return pl.pallas_call(
        _fc_body,
        grid=(batch // G,),
        in_specs=[
            pl.BlockSpec((G, C, HW), lambda i: (i, 0, 0)),
            pl.BlockSpec((C, C), lambda i: (0, 0)),
            pl.BlockSpec((C, 1), lambda i: (0, 0)),
            pl.BlockSpec((C, 1), lambda i: (0, 0)),
        ],
        out_specs=[pl.BlockSpec((G, C, HW), lambda i: (i, 0, 0)),
                   pl.BlockSpec((8, C), lambda i: (0, 0))],
        out_shape=[jax.ShapeDtypeStruct((batch, C, HW), jnp.float32),
                   jax.ShapeDtypeStruct((8, C), jnp.float32)],
    )(x, w, s, t)


def _final_body(x_ref, s_ref, t_ref, out_ref):
    x = x_ref[...].astype(jnp.float32)
    y = jnp.maximum(x * s_ref[...][None] + t_ref[...][None], 0.0)
    nrm = jnp.sqrt(jnp.sum(y * y, axis=1, keepdims=True))
    e = y / (nrm + 1e-8)
    for g in range(G):
        out_ref[g * HW:(g + 1) * HW, :] = e[g].T


def _final_layer(x, s, t):
    batch = x.shape[0]
    return pl.pallas_call(
        _final_body,
        grid=(batch // G,),
        in_specs=[
            pl.BlockSpec((G, C, HW), lambda i: (i, 0, 0)),
            pl.BlockSpec((C, 1), lambda i: (0, 0)),
            pl.BlockSpec((C, 1), lambda i: (0, 0)),
        ],
        out_specs=pl.BlockSpec((G * HW, C), lambda i: (i, 0)),
        out_shape=jax.ShapeDtypeStruct((batch * HW, C), jnp.float32),
    )(x, s, t)


def _fold_bn(stat, bias, gamma, beta, n):
    del bias
    mu0 = stat[0] / n
    var = stat[1] / n - mu0 * mu0
    scale = gamma * jax.lax.rsqrt(var + 1e-5)
    shift = beta - mu0 * scale
    return scale.reshape(C, 1), shift.reshape(C, 1)


def _split_bf16(w):
    hi = w.astype(jnp.bfloat16)
    lo = (w - hi.astype(jnp.float32)).astype(jnp.bfloat16)
    return jnp.stack([hi, lo])


def _w576(w):
    return _split_bf16(jnp.transpose(w, (2, 3, 1, 0)).reshape(9 * C, C).T)


def kernel(x, conv0a_w, conv0a_b, bn0a_g, bn0a_b, conv0b_w, conv0b_b, bn0b_g, bn0b_b,
           conv1a_w, conv1a_b, bn1a_g, bn1a_b, conv1b_w, conv1b_b, bn1b_g, bn1b_b,
           fc0_w, fc0_b, bnf0_g, bnf0_b, fc1_w, fc1_b, bnf1_g, bnf1_b):
    batch = x.shape[0]
    n = batch * HW
    x3 = x.reshape(batch, C, HW)
    ones = jnp.ones((C, 1), jnp.float32)
    zeros = jnp.zeros((C, 1), jnp.float32)

    y, st = _conv_layer(x3, _w576(conv0a_w), ones, zeros, True)
    s, t = _fold_bn(st, conv0a_b, bn0a_g, bn0a_b, n)
    y, st = _conv_layer(y, _w576(conv0b_w), s, t, False)
    s, t = _fold_bn(st, conv0b_b, bn0b_g, bn0b_b, n)
    y, st = _conv_layer(y, _w576(conv1a_w), s, t, False)
    s, t = _fold_bn(st, conv1a_b, bn1a_g, bn1a_b, n)
    y, st = _conv_layer(y, _w576(conv1b_w), s, t, False)
    s, t = _fold_bn(st, conv1b_b, bn1b_g, bn1b_b, n)

    e, st = _fc_layer(y, fc0_w, s, t)
    s, t = _fold_bn(st, fc0_b, bnf0_g, bnf0_b, n)
    e, st = _fc_layer(e, fc1_w, s, t)
    s, t = _fold_bn(st, fc1_b, bnf1_g, bnf1_b, n)

    return _final_layer(e, s, t)


# speedup vs baseline: 1.0432x; 1.0432x over previous
import functools

import jax
import jax.numpy as jnp
from jax.experimental import pallas as pl
from jax.experimental.pallas import tpu as pltpu

C = 64
H = W = 28
HW = H * W
LPAD = 128
SEG = 1024
G = 16


def _conv_body(first, x_ref, w_ref, s_ref, t_ref, out_ref, stat_ref, patch_ref):
    step = pl.program_id(0)

    @pl.when(step == 0)
    def _init():
        stat_ref[...] = jnp.zeros_like(stat_ref)
        patch_ref[...] = jnp.zeros_like(patch_ref)

    if first:
        xn = x_ref[...]
    else:
        x = x_ref[...].astype(jnp.float32)
        xn = jnp.maximum(x * s_ref[...][None] + t_ref[...][None], 0.0)

    col = jax.lax.broadcasted_iota(jnp.int32, (1, HW), 1) % W
    m_left = jnp.where(col == W - 1, 0.0, 1.0)
    m_right = jnp.where(col == 0, 0.0, 1.0)
    xb = xn.astype(jnp.bfloat16)
    xb_l = (xn * m_left).astype(jnp.bfloat16)
    xb_r = (xn * m_right).astype(jnp.bfloat16)

    for g in range(G):
        base = g * SEG + LPAD
        srcs = (xb_l[g], xb[g], xb_r[g])
        for ky in range(3):
            for kx in range(3):
                off = (ky - 1) * W + (kx - 1)
                patch_ref[ky * 3 + kx, :, base - off:base - off + HW] = srcs[kx]

    xs = patch_ref[...].reshape(9 * C, G * SEG)
    y = (jnp.dot(w_ref[0], xs, preferred_element_type=jnp.float32)
         + jnp.dot(w_ref[1], xs, preferred_element_type=jnp.float32))

    pos = jax.lax.broadcasted_iota(jnp.int32, (1, G * SEG), 1) % SEG
    y = y * jnp.where((pos >= LPAD) & (pos < LPAD + HW), 1.0, 0.0)
    stat_ref[0:1, :] = stat_ref[0:1, :] + jnp.sum(y, axis=1)[None]
    stat_ref[1:2, :] = stat_ref[1:2, :] + jnp.sum(y * y, axis=1)[None]
    for g in range(G):
        base = g * SEG + LPAD
        out_ref[g] = y[:, base:base + HW]


def _conv_layer(x, w576, s, t, first):
    batch = x.shape[0]
    return pl.pallas_call(
        functools.partial(_conv_body, first),
        grid=(batch // G,),
        in_specs=[
            pl.BlockSpec((G, C, HW), lambda i: (i, 0, 0)),
            pl.BlockSpec((2, C, 9 * C), lambda i: (0, 0, 0)),
            pl.BlockSpec((C, 1), lambda i: (0, 0)),
            pl.BlockSpec((C, 1), lambda i: (0, 0)),
        ],
        out_specs=[pl.BlockSpec((G, C, HW), lambda i: (i, 0, 0)),
                   pl.BlockSpec((8, C), lambda i: (0, 0))],
        out_shape=[jax.ShapeDtypeStruct((batch, C, HW), jnp.float32),
                   jax.ShapeDtypeStruct((8, C), jnp.float32)],
        scratch_shapes=[pltpu.VMEM((9, C, G * SEG), jnp.bfloat16)],
    )(x, w576, s, t)


def _fc_body(x_ref, w_ref, s_ref, t_ref, out_ref, stat_ref):
    step = pl.program_id(0)

    @pl.when(step == 0)
    def _init():
        stat_ref[...] = jnp.zeros_like(stat_ref)

    x = x_ref[...].astype(jnp.float32)
    xn = jnp.maximum(x * s_ref[...][None] + t_ref[...][None], 0.0)
    w = w_ref[...]
    ssum = None
    ssq = None
    for g in range(G):
        e = jnp.dot(w, xn[g], preferred_element_type=jnp.float32)
        out_ref[g] = e
        ps = jnp.sum(e, axis=1)
        pq = jnp.sum(e * e, axis=1)
        ssum = ps if ssum is None else ssum + ps
        ssq = pq if ssq is None else ssq + pq
    stat_ref[0:1, :] = stat_ref[0:1, :] + ssum[None]
    stat_ref[1:2, :] = stat_ref[1:2, :] + ssq[None]


def _fc_layer(x, w, s, t):
    batch = x.shape[0]
    return pl.pallas_call(
        _fc_body,
        grid=(batch // G,),
        in_specs=[
            pl.BlockSpec((G, C, HW), lambda i: (i, 0, 0)),
            pl.BlockSpec((C, C), lambda i: (0, 0)),
            pl.BlockSpec((C, 1), lambda i: (0, 0)),
            pl.BlockSpec((C, 1), lambda i: (0, 0)),
        ],
        out_specs=[pl.BlockSpec((G, C, HW), lambda i: (i, 0, 0)),
                   pl.BlockSpec((8, C), lambda i: (0, 0))],
        out_shape=[jax.ShapeDtypeStruct((batch, C, HW), jnp.float32),
                   jax.ShapeDtypeStruct((8, C), jnp.float32)],
    )(x, w, s, t)


def _final_body(x_ref, s_ref, t_ref, out_ref):
    x = x_ref[...].astype(jnp.float32)
    y = jnp.maximum(x * s_ref[...][None] + t_ref[...][None], 0.0)
    nrm = jnp.sqrt(jnp.sum(y * y, axis=1, keepdims=True))
    e = y / (nrm + 1e-8)
    for g in range(G):
        out_ref[g * HW:(g + 1) * HW, :] = e[g].T


def _final_layer(x, s, t):
    batch = x.shape[0]
    return pl.pallas_call(
        _final_body,
        grid=(batch // G,),
        in_specs=[
            pl.BlockSpec((G, C, HW), lambda i: (i, 0, 0)),
            pl.BlockSpec((C, 1), lambda i: (0, 0)),
            pl.BlockSpec((C, 1), lambda i: (0, 0)),
        ],
        out_specs=pl.BlockSpec((G * HW, C), lambda i: (i, 0)),
        out_shape=jax.ShapeDtypeStruct((batch * HW, C), jnp.float32),
    )(x, s, t)


def _fold_bn(stat, bias, gamma, beta, n):
    del bias
    mu0 = stat[0] / n
    var = stat[1] / n - mu0 * mu0
    scale = gamma * jax.lax.rsqrt(var + 1e-5)
    shift = beta - mu0 * scale
    return scale.reshape(C, 1), shift.reshape(C, 1)


def _split_bf16(w):
    hi = w.astype(jnp.bfloat16)
    lo = (w - hi.astype(jnp.float32)).astype(jnp.bfloat16)
    return jnp.stack([hi, lo])


def _w576(w):
    return _split_bf16(jnp.transpose(w, (2, 3, 1, 0)).reshape(9 * C, C).T)


def kernel(x, conv0a_w, conv0a_b, bn0a_g, bn0a_b, conv0b_w, conv0b_b, bn0b_g, bn0b_b,
           conv1a_w, conv1a_b, bn1a_g, bn1a_b, conv1b_w, conv1b_b, bn1b_g, bn1b_b,
           fc0_w, fc0_b, bnf0_g, bnf0_b, fc1_w, fc1_b, bnf1_g, bnf1_b):
    batch = x.shape[0]
    n = batch * HW
    x3 = x.reshape(batch, C, HW)
    ones = jnp.ones((C, 1), jnp.float32)
    zeros = jnp.zeros((C, 1), jnp.float32)

    y, st = _conv_layer(x3, _w576(conv0a_w), ones, zeros, True)
    s, t = _fold_bn(st, conv0a_b, bn0a_g, bn0a_b, n)
    y, st = _conv_layer(y, _w576(conv0b_w), s, t, False)
    s, t = _fold_bn(st, conv0b_b, bn0b_g, bn0b_b, n)
    y, st = _conv_layer(y, _w576(conv1a_w), s, t, False)
    s, t = _fold_bn(st, conv1a_b, bn1a_g, bn1a_b, n)
    y, st = _conv_layer(y, _w576(conv1b_w), s, t, False)
    s, t = _fold_bn(st, conv1b_b, bn1b_g, bn1b_b, n)

    e, st = _fc_layer(y, fc0_w, s, t)
    s, t = _fold_bn(st, fc0_b, bnf0_g, bnf0_b, n)
    e, st = _fc_layer(e, fc1_w, s, t)
    s, t = _fold_bn(st, fc1_b, bnf1_g, bnf1_b, n)

    return _final_layer(e, s, t)
